# baseline (speedup 1.0000x reference)
import functools

import numpy as np
import jax
import jax.numpy as jnp
from jax import lax
from jax.experimental import pallas as pl
from jax.experimental.pallas import tpu as pltpu
from jax.experimental.pallas import tpu_sc as plsc

N_ATTR = 100
N_VAL = 100
ROW = N_ATTR * N_VAL
HALF = ROW // 2
NSEG = N_ATTR // 2
HOUT = NSEG * 2
HCPY = 104
BATCH = 4096
OUT = 2 * N_ATTR
OUTP = 224
L = 16
NTAB = (N_VAL + L - 1) // L
NWORKER = 32
ROWS_PER = BATCH // NWORKER
HPAD = 5008
OPAD = 112
BIGF = np.float32(2.0 ** 30)


def _perm(v, idx):
    return v.at[idx].get(mode="promise_in_bounds")


def _half_compute(buf, tab0, outbuf, zbuf):
    lane = lax.iota(jnp.int32, L)
    lanef = lane.astype(jnp.float32) + zbuf[...]
    half = lane >> 1
    even = (lane & 1) == 0
    si = jnp.zeros((L,), jnp.float32)
    chunk_cache = {}

    def chunk(k):
        if k not in chunk_cache:
            chunk_cache[k] = (buf[pl.ds(L * k, L)], lanef + float(L * k))
        return chunk_cache[k]

    for s in range(NSEG):
        lo = N_VAL * s
        hi = lo + N_VAL
        k0 = lo // L
        k1 = (hi - 1) // L
        m = None
        for k in range(k0, k1 + 1):
            start = L * k
            v, avk = chunk(k)
            full = start >= lo and start + L <= hi
            slot = jnp.float32(k - k0)
            maskc = None
            if not full:
                maskc = avk >= lo if start < lo else avk < hi
            if m is None:
                if full:
                    m, mi = v, jnp.zeros((L,), jnp.float32)
                else:
                    m = jnp.where(maskc, v, -jnp.inf)
                    mi = jnp.where(maskc, 0.0, BIGF)
            else:
                pred = v > m
                if not full:
                    pred = jnp.logical_and(pred, maskc)
                    m = jnp.where(pred, v, m)
                else:
                    m = jnp.maximum(m, v)
                mi = jnp.where(pred, slot, mi)
        g = m
        for sh in (8, 4, 2, 1):
            g = jnp.maximum(g, _perm(g, lane ^ sh))
        cand = jnp.where(m == g, mi * float(L) + lanef, BIGF)
        for sh in (8, 4, 2, 1):
            cand = jnp.minimum(cand, _perm(cand, lane ^ sh))
        si = jnp.where(lane == (s % L), cand + float(L * k0 - lo), si)
        if s % L == L - 1 or s == NSEG - 1:
            t = s // L
            sii = si.astype(jnp.int32)
            c_idx = sii >> 4
            w_idx = sii & (L - 1)
            rp = _perm(tab0[0], w_idx)
            for c in range(1, NTAB):
                hit = c_idx == c
                rp = jnp.where(hit, _perm(tab0[c], w_idx), rp)
            i0 = rp & (L - 1)
            i1 = rp >> 4
            olo = jnp.where(even, _perm(i0, half), _perm(i1, half))
            outbuf[pl.ds(2 * L * t, L)] = olo
            if 2 * L * t + 2 * L <= OPAD:
                ohi = jnp.where(even, _perm(i0, half + 8), _perm(i1, half + 8))
                outbuf[pl.ds(2 * L * t + L, L)] = ohi


def _sc_body(x_hbm, wp_hbm, msg_hbm,
             bufA, bufB, wpbuf, outA, outB, zbuf,
             semA, semB, osemA, osemB):
    wid = lax.axis_index("s") * 2 + lax.axis_index("c")
    base = wid * ROWS_PER
    last = base + ROWS_PER - 1
    pltpu.sync_copy(wp_hbm, wpbuf.at[pl.ds(0, N_VAL)])
    tab0 = [wpbuf[pl.ds(L * c, L)] for c in range(NTAB)]

    dstA = bufA.at[pl.ds(0, HALF)]
    dstB = bufB.at[pl.ds(0, HALF)]

    def srcA(r):
        return x_hbm.at[r, pl.ds(0, HALF)]

    def srcB(r):
        return x_hbm.at[r, pl.ds(HALF, HALF)]

    zbuf[...] = jnp.zeros((L,), jnp.float32)
    pltpu.async_copy(srcA(base), dstA, semA)

    def row_body(i, carry):
        r = base + 2 * i
        pltpu.async_copy(srcB(r), dstB, semB)

        pltpu.make_async_copy(srcA(r), dstA, semA).wait()
        pl.when(i > 0)(
            lambda: pltpu.make_async_copy(
                outA.at[pl.ds(0, HCPY)], msg_hbm.at[r, pl.ds(0, HCPY)],
                osemA).wait())
        _half_compute(bufA, tab0, outA, zbuf)
        pltpu.async_copy(
            outA.at[pl.ds(0, HCPY)], msg_hbm.at[r, pl.ds(0, HCPY)], osemA)

        pltpu.async_copy(srcA(jnp.minimum(r + 1, last)), dstA, semA)

        pltpu.make_async_copy(srcB(r), dstB, semB).wait()
        pl.when(i > 0)(
            lambda: pltpu.make_async_copy(
                outB.at[pl.ds(0, HCPY)], msg_hbm.at[r, pl.ds(OPAD, HCPY)],
                osemB).wait())
        _half_compute(bufB, tab0, outB, zbuf)
        pltpu.async_copy(
            outB.at[pl.ds(0, HCPY)], msg_hbm.at[r, pl.ds(OPAD, HCPY)], osemB)

        r2 = r + 1
        pltpu.async_copy(srcB(r2), dstB, semB)

        pltpu.make_async_copy(srcA(r2), dstA, semA).wait()
        pltpu.make_async_copy(
            outA.at[pl.ds(0, HCPY)], msg_hbm.at[r2, pl.ds(0, HCPY)],
            osemA).wait()
        _half_compute(bufA, tab0, outA, zbuf)
        pltpu.async_copy(
            outA.at[pl.ds(0, HCPY)], msg_hbm.at[r2, pl.ds(0, HCPY)], osemA)

        pltpu.async_copy(srcA(jnp.minimum(r2 + 1, last)), dstA, semA)

        pltpu.make_async_copy(srcB(r2), dstB, semB).wait()
        pltpu.make_async_copy(
            outB.at[pl.ds(0, HCPY)], msg_hbm.at[r2, pl.ds(OPAD, HCPY)],
            osemB).wait()
        _half_compute(bufB, tab0, outB, zbuf)
        pltpu.async_copy(
            outB.at[pl.ds(0, HCPY)], msg_hbm.at[r2, pl.ds(OPAD, HCPY)], osemB)
        return carry

    lax.fori_loop(0, ROWS_PER // 2, row_body, 0)
    pltpu.make_async_copy(srcA(last), dstA, semA).wait()
    pltpu.make_async_copy(
        outA.at[pl.ds(0, HCPY)], msg_hbm.at[last, pl.ds(0, HCPY)], osemA).wait()
    pltpu.make_async_copy(
        outB.at[pl.ds(0, HCPY)], msg_hbm.at[last, pl.ds(OPAD, HCPY)], osemB).wait()


_sc_kernel = functools.partial(
    pl.kernel,
    mesh=plsc.VectorSubcoreMesh(core_axis_name="c", subcore_axis_name="s"),
    out_type=jax.ShapeDtypeStruct((BATCH, OUTP), jnp.int32),
    compiler_params=pltpu.CompilerParams(use_tc_tiling_on_sc=False),
    scratch_types=[
        pltpu.VMEM((HPAD,), jnp.float32),
        pltpu.VMEM((HPAD,), jnp.float32),
        pltpu.VMEM((NTAB * L,), jnp.int32),
        pltpu.VMEM((OPAD,), jnp.int32),
        pltpu.VMEM((OPAD,), jnp.int32),
        pltpu.VMEM((L,), jnp.float32),
        pltpu.SemaphoreType.DMA,
        pltpu.SemaphoreType.DMA,
        pltpu.SemaphoreType.DMA,
        pltpu.SemaphoreType.DMA,
    ],
)(_sc_body)


@jax.jit
def kernel(x, mapping_weight):
    wm = mapping_weight.astype(jnp.int32)
    wpack = wm[:, 0] + L * wm[:, 1]
    msgp = _sc_kernel(x, wpack)
    msg = jnp.concatenate(
        [msgp[:, :HOUT], msgp[:, OPAD:OPAD + HOUT]], axis=1)
    zeros = jnp.zeros((BATCH, OUT), dtype=jnp.float32)
    return (msg, zeros, zeros)


# speedup vs baseline: 1.0842x; 1.0842x over previous
import functools

import numpy as np
import jax
import jax.numpy as jnp
from jax import lax
from jax.experimental import pallas as pl
from jax.experimental.pallas import tpu as pltpu
from jax.experimental.pallas import tpu_sc as plsc

N_ATTR = 100
N_VAL = 100
ROW = N_ATTR * N_VAL
HALF = ROW // 2
NSEG = N_ATTR // 2
HOUT = NSEG * 2
HCPY = 104
BATCH = 4096
OUT = 2 * N_ATTR
OUTP = 224
L = 16
NTAB = (N_VAL + L - 1) // L
NWORKER = 32
ROWS_PER = BATCH // NWORKER
HPAD = 5008
OPAD = 112
BIGF = np.float32(2.0 ** 30)


def _perm(v, idx):
    return v.at[idx].get(mode="promise_in_bounds")


def _half_compute(buf, tab0, outbuf, zbuf):
    lane = lax.iota(jnp.int32, L)
    lanef = lane.astype(jnp.float32) + zbuf[...]
    half = lane >> 1
    even = (lane & 1) == 0
    si = jnp.zeros((L,), jnp.float32)
    chunk_cache = {}

    def chunk(k):
        if k not in chunk_cache:
            chunk_cache[k] = (buf[pl.ds(L * k, L)], lanef + float(L * k))
        return chunk_cache[k]

    for s in range(NSEG):
        lo = N_VAL * s
        hi = lo + N_VAL
        k0 = lo // L
        k1 = (hi - 1) // L
        m = None
        for k in range(k0, k1 + 1):
            start = L * k
            v, avk = chunk(k)
            full = start >= lo and start + L <= hi
            slot = jnp.float32(k - k0)
            maskc = None
            if not full:
                maskc = avk >= lo if start < lo else avk < hi
            if m is None:
                if full:
                    m, mi = v, jnp.zeros((L,), jnp.float32)
                else:
                    m = jnp.where(maskc, v, -jnp.inf)
                    mi = jnp.where(maskc, 0.0, BIGF)
            else:
                pred = v > m
                if not full:
                    pred = jnp.logical_and(pred, maskc)
                    m = jnp.where(pred, v, m)
                else:
                    m = jnp.maximum(m, v)
                mi = jnp.where(pred, slot, mi)
        g = m
        for sh in (8, 4, 2, 1):
            g = jnp.maximum(g, _perm(g, lane ^ sh))
        cand = jnp.where(m == g, mi * float(L) + lanef, BIGF)
        for sh in (8, 4, 2, 1):
            cand = jnp.minimum(cand, _perm(cand, lane ^ sh))
        si = jnp.where(lane == (s % L), cand + float(L * k0 - lo), si)
        if s % L == L - 1 or s == NSEG - 1:
            t = s // L
            sii = si.astype(jnp.int32)
            c_idx = sii >> 4
            w_idx = sii & (L - 1)
            rp = _perm(tab0[0], w_idx)
            for c in range(1, NTAB):
                hit = c_idx == c
                rp = jnp.where(hit, _perm(tab0[c], w_idx), rp)
            i0 = rp & (L - 1)
            i1 = rp >> 4
            olo = jnp.where(even, _perm(i0, half), _perm(i1, half))
            outbuf[pl.ds(2 * L * t, L)] = olo
            if 2 * L * t + 2 * L <= OPAD:
                ohi = jnp.where(even, _perm(i0, half + 8), _perm(i1, half + 8))
                outbuf[pl.ds(2 * L * t + L, L)] = ohi


def _sc_body(x_hbm, wp_hbm, msg_hbm,
             bufA, bufB, wpbuf, outA, outB, zbuf,
             semA, semB, osemA, osemB):
    wid = lax.axis_index("s") * 2 + lax.axis_index("c")
    base = wid * ROWS_PER
    last = base + ROWS_PER - 1
    pltpu.sync_copy(wp_hbm, wpbuf.at[pl.ds(0, N_VAL)])
    tab0 = [wpbuf[pl.ds(L * c, L)] for c in range(NTAB)]

    dstA = bufA.at[pl.ds(0, HALF)]
    dstB = bufB.at[pl.ds(0, HALF)]

    def srcA(r):
        return x_hbm.at[r, pl.ds(0, HALF)]

    def srcB(r):
        return x_hbm.at[r, pl.ds(HALF, HALF)]

    zbuf[...] = jnp.zeros((L,), jnp.float32)
    pltpu.async_copy(srcA(base), dstA, semA)

    def row_body(i, carry):
        r = base + i
        pltpu.async_copy(srcB(r), dstB, semB)

        pltpu.make_async_copy(srcA(r), dstA, semA).wait()
        pl.when(i > 0)(
            lambda: pltpu.make_async_copy(
                outA.at[pl.ds(0, HCPY)], msg_hbm.at[r, pl.ds(0, HCPY)],
                osemA).wait())
        _half_compute(bufA, tab0, outA, zbuf)
        pltpu.async_copy(
            outA.at[pl.ds(0, HCPY)], msg_hbm.at[r, pl.ds(0, HCPY)], osemA)

        pltpu.async_copy(srcA(jnp.minimum(r + 1, last)), dstA, semA)

        pltpu.make_async_copy(srcB(r), dstB, semB).wait()
        pl.when(i > 0)(
            lambda: pltpu.make_async_copy(
                outB.at[pl.ds(0, HCPY)], msg_hbm.at[r, pl.ds(OPAD, HCPY)],
                osemB).wait())
        _half_compute(bufB, tab0, outB, zbuf)
        pltpu.async_copy(
            outB.at[pl.ds(0, HCPY)], msg_hbm.at[r, pl.ds(OPAD, HCPY)], osemB)
        return carry

    lax.fori_loop(0, ROWS_PER, row_body, 0)
    pltpu.make_async_copy(srcA(last), dstA, semA).wait()
    pltpu.make_async_copy(
        outA.at[pl.ds(0, HCPY)], msg_hbm.at[last, pl.ds(0, HCPY)], osemA).wait()
    pltpu.make_async_copy(
        outB.at[pl.ds(0, HCPY)], msg_hbm.at[last, pl.ds(OPAD, HCPY)], osemB).wait()


_sc_kernel = functools.partial(
    pl.kernel,
    mesh=plsc.VectorSubcoreMesh(core_axis_name="c", subcore_axis_name="s"),
    out_type=jax.ShapeDtypeStruct((BATCH, OUTP), jnp.int32),
    compiler_params=pltpu.CompilerParams(use_tc_tiling_on_sc=False),
    scratch_types=[
        pltpu.VMEM((HPAD,), jnp.float32),
        pltpu.VMEM((HPAD,), jnp.float32),
        pltpu.VMEM((NTAB * L,), jnp.int32),
        pltpu.VMEM((OPAD,), jnp.int32),
        pltpu.VMEM((OPAD,), jnp.int32),
        pltpu.VMEM((L,), jnp.float32),
        pltpu.SemaphoreType.DMA,
        pltpu.SemaphoreType.DMA,
        pltpu.SemaphoreType.DMA,
        pltpu.SemaphoreType.DMA,
    ],
)(_sc_body)


@jax.jit
def kernel(x, mapping_weight):
    wm = mapping_weight.astype(jnp.int32)
    wpack = wm[:, 0] + L * wm[:, 1]
    msgp = _sc_kernel(x, wpack)
    msg = jnp.concatenate(
        [msgp[:, :HOUT], msgp[:, OPAD:OPAD + HOUT]], axis=1)
    zeros = jnp.zeros((BATCH, OUT), dtype=jnp.float32)
    return (msg, zeros, zeros)
